# baseline (speedup 1.0000x reference)
import functools

import jax
import jax.numpy as jnp
import numpy as np
from jax.experimental import pallas as pl
from jax.experimental.pallas import tpu as pltpu

_HIGHEST = jax.lax.Precision.HIGHEST
_LOG2PI = float(np.log(2.0 * np.pi))

_NBUF = 4
_LOOKAHEAD = 3


def _dot_t(a, b):
    return jax.lax.dot_general(a, b, (((1,), (1,)), ((), ())),
                               precision=_HIGHEST,
                               preferred_element_type=jnp.float32)


def _blk_of(j, a_steps, nb):
    jj = j - a_steps
    return jnp.where(jj < nb, jj, jj - nb)


def _fused_kernel(x_ref, adj_ref, pi_ref, mus_ref, pres_ref, w1_ref, w2_ref,
                  out_ref, probs_ref, y_ref, z_ref, buf_ref, sem_ref,
                  *, a_steps, b_steps, bm_x, bm, inv_n):
    i = pl.program_id(0)
    nb = b_steps
    total = a_steps + 2 * nb
    ab = a_steps + nb

    j = i + _LOOKAHEAD

    @pl.when((j >= a_steps) & (j < total))
    def _():
        blk = _blk_of(j, a_steps, nb)
        slot = jax.lax.rem(j, _NBUF)
        pltpu.make_async_copy(
            adj_ref.at[pl.ds(blk * bm, bm), :],
            buf_ref.at[slot],
            sem_ref.at[slot],
        ).start()

    @pl.when(i < a_steps)
    def _():
        y_ref[pl.ds(i * bm_x, bm_x), :] = jnp.dot(
            x_ref[...], w1_ref[...], preferred_element_type=jnp.float32)

    @pl.when((i >= a_steps) & (i < total))
    def _():
        blk = _blk_of(i, a_steps, nb)
        slot = jax.lax.rem(i, _NBUF)
        pltpu.make_async_copy(
            adj_ref.at[pl.ds(blk * bm, bm), :],
            buf_ref.at[slot],
            sem_ref.at[slot],
        ).wait()

    @pl.when((i >= a_steps) & (i < ab))
    def _():
        slot = jax.lax.rem(i, _NBUF)
        h = jnp.dot(buf_ref[slot], y_ref[...],
                    preferred_element_type=jnp.float32)
        pres = pres_ref[...]
        mus = mus_ref[...]
        ones = jnp.ones((1, mus.shape[1]), jnp.float32)
        cvec = (jnp.log(pi_ref[...])
                + 0.5 * _dot_t(ones, jnp.log(pres))
                - 0.5 * _dot_t(ones, mus * mus * pres)
                - 8.0 * _LOG2PI)
        ll = (_dot_t(h, mus * pres)
              - 0.5 * _dot_t(h * h, pres)
              + cvec)
        rowmax = jnp.max(ll, axis=1, keepdims=True)
        s = jnp.sum(rowmax, axis=0, keepdims=True) * (-inv_n)

        @pl.when(i == a_steps)
        def _():
            probs_ref[...] = jnp.zeros_like(probs_ref)

        probs_ref[...] = probs_ref[...] + s

        x1 = jnp.maximum(h, 0.0)
        z_ref[pl.ds((i - a_steps) * bm, bm), :] = jnp.dot(
            x1, w2_ref[...], precision=_HIGHEST,
            preferred_element_type=jnp.float32)

    @pl.when(i >= ab)
    def _():
        slot = jax.lax.rem(i, _NBUF)
        x2 = jnp.dot(buf_ref[slot], z_ref[...],
                     preferred_element_type=jnp.float32)
        m = jnp.max(x2, axis=1, keepdims=True)
        sh = x2 - m
        lse = jnp.log(jnp.sum(jnp.exp(sh), axis=1, keepdims=True))
        out_ref[...] = sh - lse


def kernel(x, adj, PI, MUs, PREs, W1, W2):
    n, d_in = x.shape
    d_hid = W1.shape[1]
    d_out = W2.shape[1]
    k = PI.shape[0]

    bm_x = 2000
    bm = 200
    a_steps = n // bm_x
    b_steps = n // bm
    ab = a_steps + b_steps

    out, probs2 = pl.pallas_call(
        functools.partial(_fused_kernel, a_steps=a_steps, b_steps=b_steps,
                          bm_x=bm_x, bm=bm, inv_n=1.0 / n),
        grid=(a_steps + 2 * b_steps,),
        in_specs=[
            pl.BlockSpec((bm_x, d_in),
                         lambda i: (jnp.minimum(i, a_steps - 1), 0)),
            pl.BlockSpec(memory_space=pltpu.MemorySpace.HBM),
            pl.BlockSpec((1, k), lambda i: (0, 0)),
            pl.BlockSpec((k, d_hid), lambda i: (0, 0)),
            pl.BlockSpec((k, d_hid), lambda i: (0, 0)),
            pl.BlockSpec((d_in, d_hid), lambda i: (0, 0)),
            pl.BlockSpec((d_hid, d_out), lambda i: (0, 0)),
        ],
        out_specs=[
            pl.BlockSpec((bm, d_out),
                         lambda i: (jnp.maximum(i - ab, 0), 0)),
            pl.BlockSpec((1, 1), lambda i: (0, 0)),
        ],
        out_shape=[
            jax.ShapeDtypeStruct((n, d_out), jnp.float32),
            jax.ShapeDtypeStruct((1, 1), jnp.float32),
        ],
        scratch_shapes=[
            pltpu.VMEM((n, d_hid), jnp.float32),
            pltpu.VMEM((n, d_out), jnp.float32),
            pltpu.VMEM((_NBUF, bm, n), jnp.float32),
            pltpu.SemaphoreType.DMA((_NBUF,)),
        ],
        compiler_params=pltpu.CompilerParams(
            dimension_semantics=("arbitrary",)),
    )(x, adj, PI.reshape(1, k), MUs, PREs, W1, W2)

    return (out, probs2[0, 0])


# speedup vs baseline: 1.0976x; 1.0976x over previous
import functools

import jax
import jax.numpy as jnp
import numpy as np
from jax.experimental import pallas as pl
from jax.experimental.pallas import tpu as pltpu

_HIGHEST = jax.lax.Precision.HIGHEST
_LOG2PI = float(np.log(2.0 * np.pi))


def _dot_t(a, b):
    return jax.lax.dot_general(a, b, (((1,), (1,)), ((), ())),
                               precision=_HIGHEST,
                               preferred_element_type=jnp.float32)


def _fused_kernel(x_ref, adj_ref, pi_ref, mus_ref, pres_ref, w1_ref, w2_ref,
                  out_ref, probs_ref, y_ref, z_ref,
                  *, a_steps, b_steps, bm_x, bm, inv_n):
    i = pl.program_id(0)

    @pl.when(i < a_steps)
    def _():
        y_ref[pl.ds(i * bm_x, bm_x), :] = jnp.dot(
            x_ref[...], w1_ref[...], preferred_element_type=jnp.float32)

    @pl.when((i >= a_steps) & (i < a_steps + b_steps))
    def _():
        h = jnp.dot(adj_ref[...].astype(jnp.bfloat16),
                    y_ref[...].astype(jnp.bfloat16),
                    preferred_element_type=jnp.float32)
        pres = pres_ref[...]
        mus = mus_ref[...]
        ones = jnp.ones((1, mus.shape[1]), jnp.float32)
        cvec = (jnp.log(pi_ref[...])
                + 0.5 * _dot_t(ones, jnp.log(pres))
                - 0.5 * _dot_t(ones, mus * mus * pres)
                - 8.0 * _LOG2PI)
        ll = (_dot_t(h, mus * pres)
              - 0.5 * _dot_t(h * h, pres)
              + cvec)
        rowmax = jnp.max(ll, axis=1, keepdims=True)
        s = jnp.sum(rowmax, axis=0, keepdims=True) * (-inv_n)

        @pl.when(i == a_steps)
        def _():
            probs_ref[...] = jnp.zeros_like(probs_ref)

        probs_ref[...] = probs_ref[...] + s

        x1 = jnp.maximum(h, 0.0)
        z_ref[pl.ds((i - a_steps) * bm, bm), :] = jnp.dot(
            x1, w2_ref[...], precision=_HIGHEST,
            preferred_element_type=jnp.float32)

    @pl.when(i >= a_steps + b_steps)
    def _():
        x2 = jnp.dot(adj_ref[...].astype(jnp.bfloat16),
                     z_ref[...].astype(jnp.bfloat16),
                     preferred_element_type=jnp.float32)
        m = jnp.max(x2, axis=1, keepdims=True)
        sh = x2 - m
        lse = jnp.log(jnp.sum(jnp.exp(sh), axis=1, keepdims=True))
        out_ref[...] = sh - lse


def kernel(x, adj, PI, MUs, PREs, W1, W2):
    n, d_in = x.shape
    d_hid = W1.shape[1]
    d_out = W2.shape[1]
    k = PI.shape[0]

    bm_x = 2000
    bm = 400
    a_steps = n // bm_x
    b_steps = n // bm
    ab = a_steps + b_steps

    out, probs2 = pl.pallas_call(
        functools.partial(_fused_kernel, a_steps=a_steps, b_steps=b_steps,
                          bm_x=bm_x, bm=bm, inv_n=1.0 / n),
        grid=(a_steps + 2 * b_steps,),
        in_specs=[
            pl.BlockSpec((bm_x, d_in),
                         lambda i: (jnp.minimum(i, a_steps - 1), 0)),
            pl.BlockSpec((bm, n),
                         lambda i: (jnp.where(
                             i < a_steps, 0,
                             jnp.where(i < ab, i - a_steps, i - ab)), 0)),
            pl.BlockSpec((1, k), lambda i: (0, 0)),
            pl.BlockSpec((k, d_hid), lambda i: (0, 0)),
            pl.BlockSpec((k, d_hid), lambda i: (0, 0)),
            pl.BlockSpec((d_in, d_hid), lambda i: (0, 0)),
            pl.BlockSpec((d_hid, d_out), lambda i: (0, 0)),
        ],
        out_specs=[
            pl.BlockSpec((bm, d_out),
                         lambda i: (jnp.maximum(i - ab, 0), 0)),
            pl.BlockSpec((1, 1), lambda i: (0, 0)),
        ],
        out_shape=[
            jax.ShapeDtypeStruct((n, d_out), jnp.float32),
            jax.ShapeDtypeStruct((1, 1), jnp.float32),
        ],
        scratch_shapes=[
            pltpu.VMEM((n, d_hid), jnp.float32),
            pltpu.VMEM((n, d_out), jnp.float32),
        ],
        compiler_params=pltpu.CompilerParams(
            dimension_semantics=("arbitrary",)),
    )(x, adj, PI.reshape(1, k), MUs, PREs, W1, W2)

    return (out, probs2[0, 0])


# speedup vs baseline: 1.1083x; 1.0097x over previous
import functools

import jax
import jax.numpy as jnp
import numpy as np
from jax.experimental import pallas as pl
from jax.experimental.pallas import tpu as pltpu

_HIGHEST = jax.lax.Precision.HIGHEST
_LOG2PI = float(np.log(2.0 * np.pi))


def _dot_t(a, b):
    return jax.lax.dot_general(a, b, (((1,), (1,)), ((), ())),
                               precision=_HIGHEST,
                               preferred_element_type=jnp.float32)


def _fused_kernel(x_ref, adj_ref, pi_ref, mus_ref, pres_ref, w1_ref, w2_ref,
                  out_ref, probs_ref, y_ref, z_ref,
                  *, a_steps, b_steps, bm_x, bm, inv_n):
    i = pl.program_id(0)

    def _out_block():
        x2 = jnp.dot(adj_ref[...].astype(jnp.bfloat16),
                     z_ref[...].astype(jnp.bfloat16),
                     preferred_element_type=jnp.float32)
        m = jnp.max(x2, axis=1, keepdims=True)
        sh = x2 - m
        lse = jnp.log(jnp.sum(jnp.exp(sh), axis=1, keepdims=True))
        out_ref[...] = sh - lse

    @pl.when(i < a_steps)
    def _():
        y_ref[pl.ds(i * bm_x, bm_x), :] = jnp.dot(
            x_ref[...], w1_ref[...], preferred_element_type=jnp.float32)

    @pl.when((i >= a_steps) & (i < a_steps + b_steps))
    def _():
        h = jnp.dot(adj_ref[...].astype(jnp.bfloat16),
                    y_ref[...].astype(jnp.bfloat16),
                    preferred_element_type=jnp.float32)
        pres = pres_ref[...]
        mus = mus_ref[...]
        ones = jnp.ones((1, mus.shape[1]), jnp.float32)
        cvec = (jnp.log(pi_ref[...])
                + 0.5 * _dot_t(ones, jnp.log(pres))
                - 0.5 * _dot_t(ones, mus * mus * pres)
                - 8.0 * _LOG2PI)
        ll = (_dot_t(h, mus * pres)
              - 0.5 * _dot_t(h * h, pres)
              + cvec)
        rowmax = jnp.max(ll, axis=1, keepdims=True)
        s = jnp.sum(rowmax, axis=0, keepdims=True) * (-inv_n)

        @pl.when(i == a_steps)
        def _():
            probs_ref[...] = jnp.zeros_like(probs_ref)

        probs_ref[...] = probs_ref[...] + s

        x1 = jnp.maximum(h, 0.0)
        z_ref[pl.ds((i - a_steps) * bm, bm), :] = jnp.dot(
            x1, w2_ref[...], precision=_HIGHEST,
            preferred_element_type=jnp.float32)

        @pl.when(i == a_steps + b_steps - 1)
        def _():
            _out_block()

    @pl.when(i >= a_steps + b_steps)
    def _():
        _out_block()


def kernel(x, adj, PI, MUs, PREs, W1, W2):
    n, d_in = x.shape
    d_hid = W1.shape[1]
    d_out = W2.shape[1]
    k = PI.shape[0]

    bm_x = 2000
    bm = 400
    a_steps = n // bm_x
    b_steps = n // bm
    ab = a_steps + b_steps

    out, probs2 = pl.pallas_call(
        functools.partial(_fused_kernel, a_steps=a_steps, b_steps=b_steps,
                          bm_x=bm_x, bm=bm, inv_n=1.0 / n),
        grid=(a_steps + 2 * b_steps - 1,),
        in_specs=[
            pl.BlockSpec((bm_x, d_in),
                         lambda i: (jnp.minimum(i, a_steps - 1), 0)),
            pl.BlockSpec((bm, n),
                         lambda i: (jnp.where(
                             i < a_steps, 0,
                             jnp.where(i < ab, i - a_steps, i - ab)), 0)),
            pl.BlockSpec((1, k), lambda i: (0, 0)),
            pl.BlockSpec((k, d_hid), lambda i: (0, 0)),
            pl.BlockSpec((k, d_hid), lambda i: (0, 0)),
            pl.BlockSpec((d_in, d_hid), lambda i: (0, 0)),
            pl.BlockSpec((d_hid, d_out), lambda i: (0, 0)),
        ],
        out_specs=[
            pl.BlockSpec((bm, d_out),
                         lambda i: (jnp.where(i < ab, b_steps - 1, i - ab),
                                    0)),
            pl.BlockSpec((1, 1), lambda i: (0, 0)),
        ],
        out_shape=[
            jax.ShapeDtypeStruct((n, d_out), jnp.float32),
            jax.ShapeDtypeStruct((1, 1), jnp.float32),
        ],
        scratch_shapes=[
            pltpu.VMEM((n, d_hid), jnp.float32),
            pltpu.VMEM((n, d_out), jnp.float32),
        ],
        compiler_params=pltpu.CompilerParams(
            dimension_semantics=("arbitrary",)),
    )(x, adj, PI.reshape(1, k), MUs, PREs, W1, W2)

    return (out, probs2[0, 0])
